# baseline (speedup 1.0000x reference)
import functools

import jax
import jax.numpy as jnp
from jax import lax
from jax.experimental import pallas as pl
from jax.experimental.pallas import tpu as pltpu
from jax.experimental.pallas import tpu_sc as plsc


def _build_glove(B, V, D):
    info = plsc.get_sparse_core_info()
    NC, NS, L = info.num_cores, info.num_subcores, info.num_lanes
    NW = NC * NS
    BPW = B // NW
    CH = 128
    NCH = BPW // CH
    NG = BPW // L

    mesh = plsc.VectorSubcoreMesh(core_axis_name="c", subcore_axis_name="s")

    @functools.partial(
        pl.kernel,
        mesh=mesh,
        compiler_params=pltpu.CompilerParams(
            needs_layout_passes=False, use_tc_tiling_on_sc=False),
        out_type=jax.ShapeDtypeStruct((B,), jnp.float32),
        scratch_types=[
            pltpu.VMEM((NCH, CH), jnp.int32),
            pltpu.VMEM((NCH, CH), jnp.int32),
            pltpu.VMEM((BPW, D), jnp.float32),
            pltpu.VMEM((BPW, D), jnp.float32),
            pltpu.VMEM((BPW,), jnp.float32),
            pltpu.VMEM((BPW,), jnp.float32),
            pltpu.VMEM((BPW,), jnp.float32),
            pltpu.SemaphoreType.DMA,
        ],
    )
    def glove(ii_hbm, jj_hbm, wi_hbm, wj_hbm, bi_hbm, bj_hbm, out_hbm,
              idx_i, idx_j, rows_i, rows_j, bv_i, bv_j, out_v, sem):
        wid = lax.axis_index("s") * NC + lax.axis_index("c")
        base = wid * BPW

        for c in range(NCH):
            pltpu.sync_copy(ii_hbm.at[pl.ds(base + c * CH, CH)], idx_i.at[c])
            pltpu.sync_copy(jj_hbm.at[pl.ds(base + c * CH, CH)], idx_j.at[c])

        copies = []
        for c in range(NCH):
            sl = pl.ds(c * CH, CH)
            copies.append(
                pltpu.async_copy(wi_hbm.at[idx_i.at[c]], rows_i.at[sl], sem))
            copies.append(
                pltpu.async_copy(wj_hbm.at[idx_j.at[c]], rows_j.at[sl], sem))
            copies.append(
                pltpu.async_copy(bi_hbm.at[idx_i.at[c]], bv_i.at[sl], sem))
            copies.append(
                pltpu.async_copy(bj_hbm.at[idx_j.at[c]], bv_j.at[sl], sem))
        for cp in copies:
            cp.wait()

        lane = lax.iota(jnp.int32, L)

        def body(g, carry):
            row_ids = g * L + lane
            acc = bv_i[pl.ds(g * L, L)] + bv_j[pl.ds(g * L, L)]
            for d in range(D):
                col = jnp.full((L,), d, jnp.int32)
                gi = plsc.load_gather(rows_i, [row_ids, col])
                gj = plsc.load_gather(rows_j, [row_ids, col])
                acc = acc + gi * gj
            out_v[pl.ds(g * L, L)] = acc
            return carry

        lax.fori_loop(0, NG, body, 0)

        pltpu.sync_copy(out_v, out_hbm.at[pl.ds(base, BPW)])

    return glove


def kernel(i_indices, j_indices, wi, wj, bi, bj):
    B = i_indices.shape[0]
    V, D = wi.shape
    glove = _build_glove(B, V, D)
    return glove(i_indices, j_indices, wi, wj,
                 bi.reshape(V), bj.reshape(V))
